# baseline (speedup 1.0000x reference)
import jax
import jax.numpy as jnp
from jax.experimental import pallas as pl

_BLOCK_ROWS = 8000


def _copy_body(u_ref, i_ref, uo_ref, io_ref):
    uo_ref[...] = u_ref[...]
    io_ref[...] = i_ref[...]


def kernel(user_emb, item_emb):
    n, d = user_emb.shape
    grid = n // _BLOCK_ROWS
    spec = pl.BlockSpec((_BLOCK_ROWS, d), lambda i: (i, 0))
    out = pl.pallas_call(
        _copy_body,
        grid=(grid,),
        in_specs=[spec, spec],
        out_specs=[spec, spec],
        out_shape=[
            jax.ShapeDtypeStruct(user_emb.shape, user_emb.dtype),
            jax.ShapeDtypeStruct(item_emb.shape, item_emb.dtype),
        ],
    )(user_emb, item_emb)
    return (out[0], out[1])


# speedup vs baseline: 1.0119x; 1.0119x over previous
import jax
import jax.numpy as jnp
from jax.experimental import pallas as pl

_WIDE = 128
_BLOCK_ROWS = 5000


def _copy_body(u_ref, i_ref, uo_ref, io_ref):
    uo_ref[...] = u_ref[...]
    io_ref[...] = i_ref[...]


def kernel(user_emb, item_emb):
    n, d = user_emb.shape
    wide_rows = n * d // _WIDE
    u = user_emb.reshape(wide_rows, _WIDE)
    it = item_emb.reshape(wide_rows, _WIDE)
    grid = wide_rows // _BLOCK_ROWS
    spec = pl.BlockSpec((_BLOCK_ROWS, _WIDE), lambda i: (i, 0))
    out = pl.pallas_call(
        _copy_body,
        grid=(grid,),
        in_specs=[spec, spec],
        out_specs=[spec, spec],
        out_shape=[
            jax.ShapeDtypeStruct(u.shape, u.dtype),
            jax.ShapeDtypeStruct(it.shape, it.dtype),
        ],
    )(u, it)
    return (out[0].reshape(n, d), out[1].reshape(n, d))
